# Initial kernel scaffold; baseline (speedup 1.0000x reference)
import jax
import jax.numpy as jnp
from jax.experimental import pallas as pl


def kernel(delta, score):
    raise NotImplementedError("write your pallas kernel here")


# speedup vs baseline: 9.7272x; 9.7272x over previous
import numpy as np
import jax
import jax.numpy as jnp
from jax.experimental import pallas as pl
from jax.experimental.pallas import tpu as pltpu

IMG_SIZE = (1024.0, 1024.0)
MAP_SIZE = (64, 64)
BASE_SIZE = 16
ANCHOR_RATIO = [0.5, 1.0, 2.0]
ANCHOR_SCALE = [8, 16, 32]
FEAT_STRIDE = 16
RPN_MIN_SIZE = 16.0
PRE_NMS_N = 6000
POST_NMS_N = 300
NMS_THRESH = 0.7

N = MAP_SIZE[0] * MAP_SIZE[1] * 9
ROWS, LANES = N // 128, 128

NEG1_BITS = int(np.float32(-1.0).view(np.int32))
DEAD = -2**31


def _anchor_consts():
    ratios = np.array(ANCHOR_RATIO, dtype=np.float32)
    scales = np.array(ANCHOR_SCALE, dtype=np.float32)
    base = np.array([0, 0, BASE_SIZE - 1, BASE_SIZE - 1], dtype=np.float32)
    w = base[2] - base[0] + 1
    h = base[3] - base[1] + 1
    x_ctr = base[0] + 0.5 * (w - 1)
    y_ctr = base[1] + 0.5 * (h - 1)
    size = w * h
    anchors = []
    for r in ratios:
        ws = np.round(np.sqrt(size / r))
        hs = np.round(ws * r)
        for s in scales:
            wss = ws * s
            hss = hs * s
            anchors.append([x_ctr - 0.5 * (wss - 1), y_ctr - 0.5 * (hss - 1),
                            x_ctr + 0.5 * (wss - 1), y_ctr + 0.5 * (hss - 1)])
    anchors = np.array(anchors, dtype=np.float32)
    shift_x = np.arange(0, MAP_SIZE[1]) * FEAT_STRIDE
    shift_y = np.arange(0, MAP_SIZE[0]) * FEAT_STRIDE
    sx, sy = np.meshgrid(shift_x, shift_y)
    shifts = np.stack([sx.ravel(), sy.ravel(), sx.ravel(), sy.ravel()],
                      axis=1).astype(np.float32)
    allb = (anchors.reshape(1, -1, 4) + shifts.reshape(-1, 1, 4)).reshape(-1, 4)
    widths = allb[:, 2] - allb[:, 0] + np.float32(1.0)
    heights = allb[:, 3] - allb[:, 1] + np.float32(1.0)
    ctr_x = allb[:, 0] + np.float32(0.5) * widths
    ctr_y = allb[:, 1] + np.float32(0.5) * heights
    rs = lambda a: a.reshape(ROWS, LANES)
    return rs(widths), rs(heights), rs(ctr_x), rs(ctr_y)


_AW, _AH, _ACX, _ACY = _anchor_consts()
_IDX2D = np.arange(N, dtype=np.int32).reshape(ROWS, LANES)


def _body(dx, dy, dw, dh, sc, aw, ah, acx, acy, idx2d,
          out_ref, x0r, y0r, x1r, y1r, arear, lkr):
    W = aw[...]
    H = ah[...]
    pcx = dx[...] * W + acx[...]
    pcy = dy[...] * H + acy[...]
    pw = jnp.exp(dw[...]) * W
    ph = jnp.exp(dh[...]) * H
    x0 = jnp.clip(pcx - 0.5 * pw, 0.0, IMG_SIZE[0])
    y0 = jnp.clip(pcy - 0.5 * ph, 0.0, IMG_SIZE[1])
    x1 = jnp.clip(pcx + 0.5 * pw, 0.0, IMG_SIZE[0])
    y1 = jnp.clip(pcy + 0.5 * ph, 0.0, IMG_SIZE[1])
    w = x1 - x0
    h = y1 - y0
    valid = (w >= RPN_MIN_SIZE) & (h >= RPN_MIN_SIZE)
    x0r[...] = x0
    y0r[...] = y0
    x1r[...] = x1
    y1r[...] = y1
    arear[...] = w * h
    idx = idx2d[...]

    ki = jnp.where(valid, jax.lax.bitcast_convert_type(sc[...], jnp.int32),
                   NEG1_BITS)

    def cnt_ge(t):
        return jnp.sum((ki >= t).astype(jnp.int32))

    n_ge0 = cnt_ge(0)

    def bit_body(i, k):
        k2 = k | (1 << (30 - i))
        return jnp.where(cnt_ge(k2) >= PRE_NMS_N, k2, k)
    kbits = jax.lax.fori_loop(0, 31, bit_body, jnp.int32(0))
    kstar = jnp.where(n_ge0 >= PRE_NMS_N, kbits, jnp.int32(NEG1_BITS))

    c_gt = jnp.sum((ki > kstar).astype(jnp.int32))
    tie = ki == kstar
    r = PRE_NMS_N - c_gt

    def idx_body(i, t):
        t2 = t | (1 << (15 - i))
        cnt = jnp.sum((tie & (idx < t2)).astype(jnp.int32))
        return jnp.where(cnt < r, t2, t)
    istar = jax.lax.fori_loop(0, 16, idx_body, jnp.int32(0))

    cand = (ki > kstar) | (tie & (idx <= istar))
    lkr[...] = jnp.where(cand, ki, DEAD)

    lane = jax.lax.broadcasted_iota(jnp.int32, (1, LANES), 1)

    def nms_body(i, _):
        lk = lkr[...]
        m = jnp.max(lk)
        any_alive = m >= NEG1_BITS
        idx_sel = jnp.min(jnp.where((lk == m), idx, N))
        rr = idx_sel // LANES
        cc = idx_sel % LANES
        sel_lane = (lane == cc)

        def pick(ref):
            return jnp.sum(jnp.where(sel_lane, ref[pl.ds(rr, 1), :], 0.0))

        bx0 = pick(x0r)
        by0 = pick(y0r)
        bx1 = pick(x1r)
        by1 = pick(y1r)
        ba = pick(arear)

        X0 = x0r[...]
        Y0 = y0r[...]
        X1 = x1r[...]
        Y1 = y1r[...]
        A = arear[...]
        iw = jnp.maximum(jnp.minimum(X1, bx1) - jnp.maximum(X0, bx0), 0.0)
        ih = jnp.maximum(jnp.minimum(Y1, by1) - jnp.maximum(Y0, by0), 0.0)
        inter = iw * ih
        iou = inter / (ba + A - inter + 1e-9)
        supp = iou > NMS_THRESH
        lkr[...] = jnp.where(any_alive & supp, DEAD, lk)

        row = jnp.where(lane == 0, bx0,
              jnp.where(lane == 1, by0,
              jnp.where(lane == 2, bx1,
              jnp.where(lane == 3, by1, 0.0))))
        out_ref[pl.ds(i, 1), :] = jnp.where(any_alive, row, 0.0)
        return 0

    jax.lax.fori_loop(0, POST_NMS_N, nms_body, 0)


def _run(dx, dy, dw, dh, sc, interpret=False):
    consts = (jnp.asarray(_AW), jnp.asarray(_AH), jnp.asarray(_ACX),
              jnp.asarray(_ACY), jnp.asarray(_IDX2D))
    return pl.pallas_call(
        _body,
        out_shape=jax.ShapeDtypeStruct((POST_NMS_N + 4, LANES), jnp.float32),
        scratch_shapes=[pltpu.VMEM((ROWS, LANES), jnp.float32)] * 5
                       + [pltpu.VMEM((ROWS, LANES), jnp.int32)],
        interpret=interpret,
    )(dx, dy, dw, dh, sc, *consts)


def kernel(delta, score):
    d = delta[0]
    rs = lambda a: a.reshape(ROWS, LANES)
    out = _run(rs(d[:, 0]), rs(d[:, 1]), rs(d[:, 2]), rs(d[:, 3]),
               rs(score[0, :, 1]))
    return out[:POST_NMS_N, :4][None]


# speedup vs baseline: 10.9026x; 1.1208x over previous
import numpy as np
import jax
import jax.numpy as jnp
from jax.experimental import pallas as pl
from jax.experimental.pallas import tpu as pltpu

IMG_SIZE = (1024.0, 1024.0)
MAP_SIZE = (64, 64)
BASE_SIZE = 16
ANCHOR_RATIO = [0.5, 1.0, 2.0]
ANCHOR_SCALE = [8, 16, 32]
FEAT_STRIDE = 16
RPN_MIN_SIZE = 16.0
PRE_NMS_N = 6000
POST_NMS_N = 300
NMS_THRESH = 0.7

N = MAP_SIZE[0] * MAP_SIZE[1] * 9
ROWS, LANES = N // 128, 128
CROWS = 6144 // 128

NEG1_BITS = int(np.float32(-1.0).view(np.int32))
DEAD = -2**31


def _anchor_consts():
    ratios = np.array(ANCHOR_RATIO, dtype=np.float32)
    scales = np.array(ANCHOR_SCALE, dtype=np.float32)
    base = np.array([0, 0, BASE_SIZE - 1, BASE_SIZE - 1], dtype=np.float32)
    w = base[2] - base[0] + 1
    h = base[3] - base[1] + 1
    x_ctr = base[0] + 0.5 * (w - 1)
    y_ctr = base[1] + 0.5 * (h - 1)
    size = w * h
    anchors = []
    for r in ratios:
        ws = np.round(np.sqrt(size / r))
        hs = np.round(ws * r)
        for s in scales:
            wss = ws * s
            hss = hs * s
            anchors.append([x_ctr - 0.5 * (wss - 1), y_ctr - 0.5 * (hss - 1),
                            x_ctr + 0.5 * (wss - 1), y_ctr + 0.5 * (hss - 1)])
    anchors = np.array(anchors, dtype=np.float32)
    shift_x = np.arange(0, MAP_SIZE[1]) * FEAT_STRIDE
    shift_y = np.arange(0, MAP_SIZE[0]) * FEAT_STRIDE
    sx, sy = np.meshgrid(shift_x, shift_y)
    shifts = np.stack([sx.ravel(), sy.ravel(), sx.ravel(), sy.ravel()],
                      axis=1).astype(np.float32)
    allb = (anchors.reshape(1, -1, 4) + shifts.reshape(-1, 1, 4)).reshape(-1, 4)
    widths = allb[:, 2] - allb[:, 0] + np.float32(1.0)
    heights = allb[:, 3] - allb[:, 1] + np.float32(1.0)
    ctr_x = allb[:, 0] + np.float32(0.5) * widths
    ctr_y = allb[:, 1] + np.float32(0.5) * heights
    rs = lambda a: a.reshape(ROWS, LANES)
    return rs(widths), rs(heights), rs(ctr_x), rs(ctr_y)


_AW, _AH, _ACX, _ACY = _anchor_consts()
_IDX2D = np.arange(N, dtype=np.int32).reshape(ROWS, LANES)


def _body(dx, dy, dw, dh, sc, aw, ah, acx, acy, idx2d,
          out_ref, x0r, y0r, x1r, y1r, arear, lkr):
    W = aw[...]
    H = ah[...]
    pcx = dx[...] * W + acx[...]
    pcy = dy[...] * H + acy[...]
    pw = jnp.exp(dw[...]) * W
    ph = jnp.exp(dh[...]) * H
    x0 = jnp.clip(pcx - 0.5 * pw, 0.0, IMG_SIZE[0])
    y0 = jnp.clip(pcy - 0.5 * ph, 0.0, IMG_SIZE[1])
    x1 = jnp.clip(pcx + 0.5 * pw, 0.0, IMG_SIZE[0])
    y1 = jnp.clip(pcy + 0.5 * ph, 0.0, IMG_SIZE[1])
    w = x1 - x0
    h = y1 - y0
    valid = (w >= RPN_MIN_SIZE) & (h >= RPN_MIN_SIZE)
    idx = idx2d[...]

    ki = jnp.where(valid, jax.lax.bitcast_convert_type(sc[...], jnp.int32),
                   NEG1_BITS)

    def cnt_ge(t):
        return jnp.sum((ki >= t).astype(jnp.int32))

    n_ge0 = cnt_ge(0)

    def bit_body(i, k):
        k2 = k | (1 << (30 - i))
        return jnp.where(cnt_ge(k2) >= PRE_NMS_N, k2, k)
    kbits = jax.lax.fori_loop(0, 31, bit_body, jnp.int32(0))
    kstar = jnp.where(n_ge0 >= PRE_NMS_N, kbits, jnp.int32(NEG1_BITS))

    c_gt = jnp.sum((ki > kstar).astype(jnp.int32))
    tie = ki == kstar
    r = PRE_NMS_N - c_gt

    def idx_body(i, t):
        t2 = t | (1 << (15 - i))
        cnt = jnp.sum((tie & (idx < t2)).astype(jnp.int32))
        return jnp.where(cnt < r, t2, t)
    istar = jax.lax.fori_loop(0, 16, idx_body, jnp.int32(0))

    cand = (ki > kstar) | (tie & (idx <= istar))
    lk0 = jnp.where(cand, ki, DEAD)

    c = cand.astype(jnp.int32)
    s = c
    for k in (1, 2, 4, 8, 16, 32, 64):
        s = s + jnp.concatenate(
            [jnp.zeros((ROWS, k), jnp.int32), s[:, :LANES - k]], axis=1)
    row_tot = s[:, LANES - 1:LANES]
    e = jnp.concatenate([jnp.zeros((1, 1), jnp.int32), row_tot[:-1]], axis=0)
    for k in (1, 2, 4, 8, 16, 32, 64, 128, 256):
        e = e + jnp.concatenate(
            [jnp.zeros((k, 1), jnp.int32), e[:ROWS - k]], axis=0)
    rank_incl = s + e
    dist = jnp.where(cand, idx - (rank_incl - 1), 0)

    def shflat(a, sft):
        if sft % LANES == 0:
            m = sft // LANES
            return jnp.concatenate(
                [a[m:, :], jnp.zeros((m, LANES), a.dtype)], axis=0)
        nxt = jnp.concatenate(
            [a[1:, :], jnp.zeros((1, LANES), a.dtype)], axis=0)
        return jnp.concatenate([a[:, sft:], nxt[:, :sft]], axis=1)

    vals = [x0, y0, x1, y1, w * h, lk0]
    for k in range(16):
        sft = 1 << k
        dsh = shflat(dist, sft)
        mover = ((dsh >> k) & 1) == 1
        vals = [jnp.where(mover, shflat(v, sft), v) for v in vals]
        dist = jnp.where(mover, dsh, dist)

    x0r[...] = vals[0][:CROWS]
    y0r[...] = vals[1][:CROWS]
    x1r[...] = vals[2][:CROWS]
    y1r[...] = vals[3][:CROWS]
    arear[...] = vals[4][:CROWS]
    cidx = idx[:CROWS]
    lkr[...] = jnp.where(cidx < PRE_NMS_N, vals[5][:CROWS], DEAD)

    lane = jax.lax.broadcasted_iota(jnp.int32, (1, LANES), 1)

    def nms_body(i, _):
        lk = lkr[...]
        m = jnp.max(lk)
        any_alive = m >= NEG1_BITS
        idx_sel = jnp.min(jnp.where((lk == m), cidx, N))
        rr = idx_sel // LANES
        cc = idx_sel % LANES
        sel_lane = (lane == cc)

        def pick(ref):
            return jnp.sum(jnp.where(sel_lane, ref[pl.ds(rr, 1), :], 0.0))

        bx0 = pick(x0r)
        by0 = pick(y0r)
        bx1 = pick(x1r)
        by1 = pick(y1r)
        ba = pick(arear)

        X0 = x0r[...]
        Y0 = y0r[...]
        X1 = x1r[...]
        Y1 = y1r[...]
        A = arear[...]
        iw = jnp.maximum(jnp.minimum(X1, bx1) - jnp.maximum(X0, bx0), 0.0)
        ih = jnp.maximum(jnp.minimum(Y1, by1) - jnp.maximum(Y0, by0), 0.0)
        inter = iw * ih
        iou = inter / (ba + A - inter + 1e-9)
        supp = iou > NMS_THRESH
        lkr[...] = jnp.where(any_alive & supp, DEAD, lk)

        row = jnp.where(lane == 0, bx0,
              jnp.where(lane == 1, by0,
              jnp.where(lane == 2, bx1,
              jnp.where(lane == 3, by1, 0.0))))
        out_ref[pl.ds(i, 1), :] = jnp.where(any_alive, row, 0.0)
        return 0

    jax.lax.fori_loop(0, POST_NMS_N, nms_body, 0)


def _run(dx, dy, dw, dh, sc, interpret=False):
    consts = (jnp.asarray(_AW), jnp.asarray(_AH), jnp.asarray(_ACX),
              jnp.asarray(_ACY), jnp.asarray(_IDX2D))
    return pl.pallas_call(
        _body,
        out_shape=jax.ShapeDtypeStruct((POST_NMS_N + 4, LANES), jnp.float32),
        scratch_shapes=[pltpu.VMEM((CROWS, LANES), jnp.float32)] * 5
                       + [pltpu.VMEM((CROWS, LANES), jnp.int32)],
        interpret=interpret,
    )(dx, dy, dw, dh, sc, *consts)


def kernel(delta, score):
    d = delta[0]
    rs = lambda a: a.reshape(ROWS, LANES)
    out = _run(rs(d[:, 0]), rs(d[:, 1]), rs(d[:, 2]), rs(d[:, 3]),
               rs(score[0, :, 1]))
    return out[:POST_NMS_N, :4][None]
